# Initial kernel scaffold; baseline (speedup 1.0000x reference)
import jax
import jax.numpy as jnp
from jax.experimental import pallas as pl


def kernel(x, edge_index, W1, b1, W2, b2):
    raise NotImplementedError("write your pallas kernel here")


# speedup vs baseline: 13.1619x; 13.1619x over previous
import functools

import jax
import jax.numpy as jnp
from jax import lax
from jax.experimental import pallas as pl
from jax.experimental.pallas import tpu as pltpu
from jax.experimental.pallas import tpu_sc as plsc

N = 10000
D = 128
E = 320000

NC = 2
NS = 16
NW = NC * NS

NP = 10240
EDGES_PER_W = E // NW
CHUNK = 80
NCHUNK = EDGES_PER_W // CHUNK
ROWS_PER_TILE = NP // NS


def _worker_id():
    return lax.axis_index("c") * NS + lax.axis_index("s")


def _deg_body(dst_hbm, ones_hbm, zeros1_hbm, deg_out_hbm, idx_v, ones_v, deg_sh):
    cid = lax.axis_index("c")
    sid = lax.axis_index("s")
    wid = cid * NS + sid
    rbase = sid * ROWS_PER_TILE
    pltpu.sync_copy(zeros1_hbm.at[pl.ds(rbase, ROWS_PER_TILE)],
                    deg_sh.at[pl.ds(rbase, ROWS_PER_TILE)])
    pltpu.sync_copy(ones_hbm, ones_v)
    plsc.subcore_barrier()
    ebase = wid * EDGES_PER_W

    def body(j, _):
        st = pl.multiple_of(ebase + j * CHUNK, 8)
        pltpu.sync_copy(dst_hbm.at[pl.ds(st, CHUNK)], idx_v)
        pltpu.sync_copy(ones_v, deg_sh.at[idx_v], add=True)
        return ()

    lax.fori_loop(0, NCHUNK, body, ())
    plsc.subcore_barrier()
    pltpu.sync_copy(deg_sh.at[pl.ds(rbase, ROWS_PER_TILE)],
                    deg_out_hbm.at[pl.ds(cid * NP + rbase, ROWS_PER_TILE)])


def _agg_body(h_hbm, src_hbm, dst_hbm, zeros_hbm, out_hbm,
              sidx_v, didx_v, rows_v, agg_sh, sem):
    cid = lax.axis_index("c")
    sid = lax.axis_index("s")
    wid = cid * NS + sid
    rbase = sid * ROWS_PER_TILE
    pltpu.sync_copy(zeros_hbm.at[pl.ds(rbase, ROWS_PER_TILE)],
                    agg_sh.at[pl.ds(rbase, ROWS_PER_TILE)])
    plsc.subcore_barrier()
    ebase = wid * EDGES_PER_W

    def body(j, _):
        st = pl.multiple_of(ebase + j * CHUNK, 8)
        pltpu.sync_copy(src_hbm.at[pl.ds(st, CHUNK)], sidx_v)
        pltpu.sync_copy(dst_hbm.at[pl.ds(st, CHUNK)], didx_v)
        pltpu.async_copy(h_hbm.at[sidx_v], rows_v, sem).wait()
        pltpu.sync_copy(rows_v, agg_sh.at[didx_v], add=True)
        return ()

    lax.fori_loop(0, NCHUNK, body, ())
    plsc.subcore_barrier()
    pltpu.sync_copy(agg_sh.at[pl.ds(rbase, ROWS_PER_TILE)],
                    out_hbm.at[pl.ds(cid * NP + rbase, ROWS_PER_TILE)])


def _sc_calls():
    mesh = plsc.VectorSubcoreMesh(core_axis_name="c", subcore_axis_name="s")
    deg_call = pl.kernel(
        _deg_body,
        out_type=jax.ShapeDtypeStruct((NC * NP,), jnp.float32),
        mesh=mesh,
        scratch_types=[
            pltpu.VMEM((CHUNK,), jnp.int32),
            pltpu.VMEM((CHUNK,), jnp.float32),
            pltpu.VMEM_SHARED((NP,), jnp.float32),
        ],
    )
    agg_call = pl.kernel(
        _agg_body,
        out_type=jax.ShapeDtypeStruct((NC * NP, D), jnp.float32),
        mesh=mesh,
        scratch_types=[
            pltpu.VMEM((CHUNK,), jnp.int32),
            pltpu.VMEM((CHUNK,), jnp.int32),
            pltpu.VMEM((CHUNK, D), jnp.float32),
            pltpu.VMEM_SHARED((NP, D), jnp.float32),
            pltpu.SemaphoreType.DMA,
        ],
    )
    return deg_call, agg_call


def _stage_a_body(x_ref, w1_ref, degp_ref, h1p_ref, d_ref):
    dp = degp_ref[...]
    deg = dp[:N] + dp[NP:NP + N] + 1.0
    d = lax.rsqrt(deg)
    h = jnp.dot(x_ref[...], w1_ref[...], preferred_element_type=jnp.float32)
    h1p_ref[...] = h * d
    d_ref[...] = d


def _stage_b_body(s_ref, h1p_ref, d_ref, b1_ref, w2_ref, h2p_ref):
    s = s_ref[...]
    h1p = h1p_ref[...]
    d = d_ref[...]
    agg = s[:N] + s[NP:NP + N] + h1p
    out1 = jnp.tanh(agg * d + b1_ref[...])
    h2 = jnp.dot(out1, w2_ref[...], preferred_element_type=jnp.float32)
    h2p_ref[...] = h2 * d


def _stage_c_body(s_ref, h2p_ref, d_ref, b2_ref, out_ref):
    s = s_ref[...]
    agg = s[:N] + s[NP:NP + N] + h2p_ref[...]
    out_ref[...] = agg * d_ref[...] + b2_ref[...]


def kernel(x, edge_index, W1, b1, W2, b2):
    src = edge_index[0]
    dst = edge_index[1]
    zeros2 = jnp.zeros((NP, D), jnp.float32)
    zeros1 = jnp.zeros((NP,), jnp.float32)
    ones = jnp.ones((CHUNK,), jnp.float32)
    b1r = b1.reshape(1, D)
    b2r = b2.reshape(1, D)

    deg_call, agg_call = _sc_calls()

    degp = deg_call(dst, ones, zeros1).reshape(NC * NP, 1)

    h1p, d = pl.pallas_call(
        _stage_a_body,
        out_shape=(jax.ShapeDtypeStruct((N, D), jnp.float32),
                   jax.ShapeDtypeStruct((N, 1), jnp.float32)),
    )(x, W1, degp)

    s1 = agg_call(h1p, src, dst, zeros2)

    h2p = pl.pallas_call(
        _stage_b_body,
        out_shape=jax.ShapeDtypeStruct((N, D), jnp.float32),
    )(s1, h1p, d, b1r, W2)

    s2 = agg_call(h2p, src, dst, zeros2)

    out = pl.pallas_call(
        _stage_c_body,
        out_shape=jax.ShapeDtypeStruct((N, D), jnp.float32),
    )(s2, h2p, d, b2r)
    return out
